# baseline (speedup 1.0000x reference)
import jax
import jax.numpy as jnp
from jax.experimental import pallas as pl
from jax.experimental.pallas import tpu as pltpu

_PATCH = 14
_D = 128
_DEPTH = 2
_HD = 64
_EPS = 1e-6
_S = 197
_SP = 200
_BB = 8
_NEG = -1e30


def _ln(v, w, b):
    mu = jnp.mean(v, axis=-1, keepdims=True)
    var = jnp.mean(v * v, axis=-1, keepdims=True) - mu * mu
    return (v - mu) * jax.lax.rsqrt(var + _EPS) * w + b


def _fused_kernel(patches, pw, add_init,
                  ln1w, ln1b, wqkv, bqkv, wp, bp, g1,
                  ln2w, ln2b, w12, b12, w3, b3, g2,
                  nw, nb, o_ref):
    R = _BB * _SP
    bf16 = jnp.bfloat16
    f32 = jnp.float32

    tok = jnp.dot(patches[...], pw[...], preferred_element_type=f32)
    x = (tok.reshape(_BB, _SP, _D) + add_init[...]).reshape(R, _D)

    lane = jax.lax.broadcasted_iota(jnp.int32, (R, _D), 1)
    head0 = lane < _HD
    col = jax.lax.broadcasted_iota(jnp.int32, (_SP, _SP), 1)
    kvalid = col < _S
    scale = _HD ** -0.5

    for l in range(_DEPTH):
        h = _ln(x, ln1w[l], ln1b[l]).astype(bf16)
        qkv = jnp.dot(h, wqkv[l], preferred_element_type=f32) + bqkv[l]
        q = qkv[:, :_D] * scale
        k = qkv[:, _D:2 * _D]
        v = qkv[:, 2 * _D:]
        qb = q.astype(bf16)
        k0 = jnp.where(head0, k, 0.0).astype(bf16)
        k1 = jnp.where(head0, 0.0, k).astype(bf16)
        v0 = jnp.where(head0, v, 0.0).astype(bf16)
        v1 = jnp.where(head0, 0.0, v).astype(bf16)

        outs = []
        for i in range(_BB):
            a = i * _SP
            qi = qb[a:a + _SP]
            oi = None
            for kk, vv in ((k0, v0), (k1, v1)):
                s = jax.lax.dot_general(
                    qi, kk[a:a + _SP],
                    (((1,), (1,)), ((), ())), preferred_element_type=f32)
                s = jnp.where(kvalid, s, _NEG)
                s = s - jnp.max(s, axis=-1, keepdims=True)
                e = jnp.exp(s)
                p = e * pl.reciprocal(jnp.sum(e, axis=-1, keepdims=True),
                                      approx=True)
                pv = jnp.dot(p.astype(bf16), vv[a:a + _SP],
                             preferred_element_type=f32)
                oi = pv if oi is None else oi + pv
            outs.append(oi)
        attn = jnp.concatenate(outs, axis=0)

        pr = jnp.dot(attn.astype(bf16), wp[l], preferred_element_type=f32) + bp[l]
        x = x + pr * g1[l]

        h2 = _ln(x, ln2w[l], ln2b[l]).astype(bf16)
        x12 = jnp.dot(h2, w12[l], preferred_element_type=f32) + b12[l]
        hw = w12.shape[-1] // 2
        x1 = x12[:, :hw]
        x2 = x12[:, hw:]
        sig = pl.reciprocal(1.0 + jnp.exp(-x1), approx=True)
        hid = ((x1 * sig) * x2).astype(bf16)
        ml = jnp.dot(hid, w3[l], preferred_element_type=f32) + b3[l]
        x = x + ml * g2[l]

    cls = jnp.concatenate([x[i * _SP:i * _SP + 1] for i in range(_BB)], axis=0)
    o_ref[...] = _ln(cls, nw[...], nb[...])


def kernel(x, patch_w, patch_b, cls_token, pos_embed, norm_w, norm_b,
           blk00, blk01, blk02, blk03, blk04, blk05, blk06, blk07, blk08,
           blk09, blk10, blk11, blk12, blk13, blk14, blk15, blk16, blk17,
           blk18, blk19):
    B, C, H, W = x.shape
    Hn, Wn = H // _PATCH, W // _PATCH
    np_ = Hn * Wn
    k_raw = C * _PATCH * _PATCH
    bf16 = jnp.bfloat16
    f32 = jnp.float32

    patches = jax.lax.reshape(
        x.astype(bf16).reshape(B, C, Hn, _PATCH, Wn, _PATCH),
        (B, np_, k_raw), dimensions=(0, 2, 4, 1, 3, 5))
    patches = jnp.pad(patches, ((0, 0), (1, _SP - 1 - np_), (0, 0)))
    patches = patches.reshape(B * _SP, k_raw)

    pw = patch_w[:k_raw].astype(bf16)

    add_init = jnp.concatenate([
        cls_token.reshape(1, _D) + pos_embed[0, 0:1],
        patch_b + pos_embed[0, 1:],
        jnp.zeros((_SP - 1 - np_, _D), f32)], axis=0)

    wqkv = jnp.concatenate([blk02, blk04, blk06], axis=2).astype(bf16)
    bqkv = jnp.concatenate([blk03, blk05, blk07], axis=2)
    wp = blk08.astype(bf16)
    w12 = jnp.concatenate([blk13, blk15], axis=2).astype(bf16)
    b12 = jnp.concatenate([blk14, blk16], axis=2)
    w3 = blk17.astype(bf16)

    def const(shape):
        nd = len(shape)
        return pl.BlockSpec(shape, lambda g, _n=nd: (0,) * _n)

    R = _BB * _SP
    args = (patches, pw, add_init,
            blk00, blk01, wqkv, bqkv, wp, blk09, blk10,
            blk11, blk12, w12, b12, w3, blk18, blk19,
            norm_w, norm_b)
    in_specs = [pl.BlockSpec((R, k_raw), lambda g: (g, 0))] + \
               [const(a.shape) for a in args[1:]]

    out = pl.pallas_call(
        _fused_kernel,
        grid=(B // _BB,),
        out_shape=jax.ShapeDtypeStruct((B, _D), f32),
        in_specs=in_specs,
        out_specs=pl.BlockSpec((_BB, _D), lambda g: (g, 0)),
        compiler_params=pltpu.CompilerParams(dimension_semantics=("parallel",)),
    )(*args)
    return out


# speedup vs baseline: 1.1944x; 1.1944x over previous
import jax
import jax.numpy as jnp
from jax.experimental import pallas as pl
from jax.experimental.pallas import tpu as pltpu

_PATCH = 14
_D = 128
_DEPTH = 2
_HD = 64
_EPS = 1e-6
_S = 197
_SP = 200
_BB = 8
_NEG = -1e30


def _ln(v, w, b):
    mu = jnp.mean(v, axis=-1, keepdims=True)
    var = jnp.mean(v * v, axis=-1, keepdims=True) - mu * mu
    return (v - mu) * jax.lax.rsqrt(var + _EPS) * w + b


def _fused_kernel(patches, pw, add_init,
                  ln1w, ln1b, wqkv, bqkv, wp, bp, g1,
                  ln2w, ln2b, w12, b12, w3, b3, g2,
                  nw, nb, o_ref):
    R = _BB * _SP
    bf16 = jnp.bfloat16
    f32 = jnp.float32

    tok = jnp.dot(patches[...], pw[...], preferred_element_type=f32)
    x = (tok.reshape(_BB, _SP, _D) + add_init[...]).reshape(R, _D)

    lane = jax.lax.broadcasted_iota(jnp.int32, (R, _D), 1)
    head0 = lane < _HD
    col = jax.lax.broadcasted_iota(jnp.int32, (_SP, _SP), 1)
    kvalid = col < _S
    scale = _HD ** -0.5

    for l in range(_DEPTH):
        h = _ln(x, ln1w[l], ln1b[l]).astype(bf16)
        qkv = jnp.dot(h, wqkv[l], preferred_element_type=f32) + bqkv[l]
        q = qkv[:, :_D] * scale
        k = qkv[:, _D:2 * _D]
        v = qkv[:, 2 * _D:]
        qb = q.astype(bf16)
        k0 = jnp.where(head0, k, 0.0).astype(bf16)
        k1 = jnp.where(head0, 0.0, k).astype(bf16)
        v0 = jnp.where(head0, v, 0.0).astype(bf16)
        v1 = jnp.where(head0, 0.0, v).astype(bf16)

        outs = []
        for i in range(_BB):
            a = i * _SP
            qi = qb[a:a + _SP]
            oi = None
            for kk, vv in ((k0, v0), (k1, v1)):
                s = jax.lax.dot_general(
                    qi, kk[a:a + _SP],
                    (((1,), (1,)), ((), ())), preferred_element_type=f32)
                s = jnp.where(kvalid, s, _NEG)
                s = s - jnp.max(s, axis=-1, keepdims=True)
                e = jnp.exp(s)
                p = e * pl.reciprocal(jnp.sum(e, axis=-1, keepdims=True),
                                      approx=True)
                pv = jnp.dot(p.astype(bf16), vv[a:a + _SP],
                             preferred_element_type=f32)
                oi = pv if oi is None else oi + pv
            outs.append(oi)
        attn = jnp.concatenate(outs, axis=0)

        pr = jnp.dot(attn.astype(bf16), wp[l], preferred_element_type=f32) + bp[l]
        x = x + pr * g1[l]

        h2 = _ln(x, ln2w[l], ln2b[l]).astype(bf16)
        x12 = jnp.dot(h2, w12[l], preferred_element_type=f32) + b12[l]
        hw = w12.shape[-1] // 2
        x1 = x12[:, :hw]
        x2 = x12[:, hw:]
        sig = pl.reciprocal(1.0 + jnp.exp(-x1), approx=True)
        hid = ((x1 * sig) * x2).astype(bf16)
        ml = jnp.dot(hid, w3[l], preferred_element_type=f32) + b3[l]
        x = x + ml * g2[l]

    cls = jnp.concatenate([x[i * _SP:i * _SP + 1] for i in range(_BB)], axis=0)
    o_ref[...] = _ln(cls, nw[...], nb[...])


def kernel(x, patch_w, patch_b, cls_token, pos_embed, norm_w, norm_b,
           blk00, blk01, blk02, blk03, blk04, blk05, blk06, blk07, blk08,
           blk09, blk10, blk11, blk12, blk13, blk14, blk15, blk16, blk17,
           blk18, blk19):
    B, C, H, W = x.shape
    Hn, Wn = H // _PATCH, W // _PATCH
    np_ = Hn * Wn
    k_raw = C * _PATCH * _PATCH
    bf16 = jnp.bfloat16
    f32 = jnp.float32

    patches = x.astype(bf16).transpose(0, 2, 3, 1)
    patches = patches.reshape(B, Hn, _PATCH, Wn, _PATCH * C)
    patches = patches.transpose(0, 1, 3, 2, 4)
    patches = patches.reshape(B, np_, k_raw)
    patches = jnp.pad(patches, ((0, 0), (1, _SP - 1 - np_), (0, 0)))
    patches = patches.reshape(B * _SP, k_raw)

    pw = patch_w[:k_raw].reshape(C, _PATCH, _PATCH, _D)
    pw = pw.transpose(1, 2, 0, 3).reshape(k_raw, _D).astype(bf16)

    add_init = jnp.concatenate([
        cls_token.reshape(1, _D) + pos_embed[0, 0:1],
        patch_b + pos_embed[0, 1:],
        jnp.zeros((_SP - 1 - np_, _D), f32)], axis=0)

    wqkv = jnp.concatenate([blk02, blk04, blk06], axis=2).astype(bf16)
    bqkv = jnp.concatenate([blk03, blk05, blk07], axis=2)
    wp = blk08.astype(bf16)
    w12 = jnp.concatenate([blk13, blk15], axis=2).astype(bf16)
    b12 = jnp.concatenate([blk14, blk16], axis=2)
    w3 = blk17.astype(bf16)

    def const(shape):
        nd = len(shape)
        return pl.BlockSpec(shape, lambda g, _n=nd: (0,) * _n)

    R = _BB * _SP
    args = (patches, pw, add_init,
            blk00, blk01, wqkv, bqkv, wp, blk09, blk10,
            blk11, blk12, w12, b12, w3, blk18, blk19,
            norm_w, norm_b)
    in_specs = [pl.BlockSpec((R, k_raw), lambda g: (g, 0))] + \
               [const(a.shape) for a in args[1:]]

    out = pl.pallas_call(
        _fused_kernel,
        grid=(B // _BB,),
        out_shape=jax.ShapeDtypeStruct((B, _D), f32),
        in_specs=in_specs,
        out_specs=pl.BlockSpec((_BB, _D), lambda g: (g, 0)),
        compiler_params=pltpu.CompilerParams(dimension_semantics=("parallel",)),
    )(*args)
    return out


# speedup vs baseline: 1.2587x; 1.0538x over previous
import jax
import jax.numpy as jnp
from jax.experimental import pallas as pl
from jax.experimental.pallas import tpu as pltpu

_PATCH = 14
_D = 128
_DEPTH = 2
_HD = 64
_EPS = 1e-6
_S = 197
_SP = 200
_NP = 196
_BB = 8
_NEG = -1e30


def _ln(v, w, b):
    mu = jnp.mean(v, axis=-1, keepdims=True)
    var = jnp.mean(v * v, axis=-1, keepdims=True) - mu * mu
    return (v - mu) * jax.lax.rsqrt(var + _EPS) * w + b


def _fused_kernel(patches, pw, add_init,
                  ln1w, ln1b, wqkv, bqkv, wp, bp, g1,
                  ln2w, ln2b, w12, b12, w3, b3, g2,
                  nw, nb, o_ref):
    R = _BB * _SP
    bf16 = jnp.bfloat16
    f32 = jnp.float32

    tok = jnp.dot(patches[...], pw[...], preferred_element_type=f32)
    a_cls = add_init[0:1]
    a_pat = add_init[8:8 + _NP]
    zpad = jnp.zeros((_SP - 1 - _NP, _D), f32)
    pieces = []
    for i in range(_BB):
        pieces += [a_cls, tok[i * _NP:(i + 1) * _NP] + a_pat, zpad]
    x = jnp.concatenate(pieces, axis=0)

    lane = jax.lax.broadcasted_iota(jnp.int32, (R, _D), 1)
    head0 = lane < _HD
    col = jax.lax.broadcasted_iota(jnp.int32, (_SP, _SP), 1)
    kvalid = col < _S
    scale = _HD ** -0.5

    for l in range(_DEPTH):
        h = _ln(x, ln1w[l], ln1b[l]).astype(bf16)
        qkv = jnp.dot(h, wqkv[l], preferred_element_type=f32) + bqkv[l]
        q = qkv[:, :_D] * scale
        k = qkv[:, _D:2 * _D]
        v = qkv[:, 2 * _D:]
        qb = q.astype(bf16)
        k0 = jnp.where(head0, k, 0.0).astype(bf16)
        k1 = jnp.where(head0, 0.0, k).astype(bf16)
        v0 = jnp.where(head0, v, 0.0).astype(bf16)
        v1 = jnp.where(head0, 0.0, v).astype(bf16)

        outs = []
        for i in range(_BB):
            a = i * _SP
            qi = qb[a:a + _SP]
            oi = None
            for kk, vv in ((k0, v0), (k1, v1)):
                s = jax.lax.dot_general(
                    qi, kk[a:a + _SP],
                    (((1,), (1,)), ((), ())), preferred_element_type=f32)
                s = jnp.where(kvalid, s, _NEG)
                s = s - jnp.max(s, axis=-1, keepdims=True)
                e = jnp.exp(s)
                p = e * pl.reciprocal(jnp.sum(e, axis=-1, keepdims=True),
                                      approx=True)
                pv = jnp.dot(p.astype(bf16), vv[a:a + _SP],
                             preferred_element_type=f32)
                oi = pv if oi is None else oi + pv
            outs.append(oi)
        attn = jnp.concatenate(outs, axis=0)

        pr = jnp.dot(attn.astype(bf16), wp[l], preferred_element_type=f32) + bp[l]
        x = x + pr * g1[l]

        h2 = _ln(x, ln2w[l], ln2b[l]).astype(bf16)
        x12 = jnp.dot(h2, w12[l], preferred_element_type=f32) + b12[l]
        hw = w12.shape[-1] // 2
        x1 = x12[:, :hw]
        x2 = x12[:, hw:]
        sig = pl.reciprocal(1.0 + jnp.exp(-x1), approx=True)
        hid = ((x1 * sig) * x2).astype(bf16)
        ml = jnp.dot(hid, w3[l], preferred_element_type=f32) + b3[l]
        x = x + ml * g2[l]

    cls = jnp.concatenate([x[i * _SP:i * _SP + 1] for i in range(_BB)], axis=0)
    o_ref[...] = _ln(cls, nw[...], nb[...])


def kernel(x, patch_w, patch_b, cls_token, pos_embed, norm_w, norm_b,
           blk00, blk01, blk02, blk03, blk04, blk05, blk06, blk07, blk08,
           blk09, blk10, blk11, blk12, blk13, blk14, blk15, blk16, blk17,
           blk18, blk19):
    B, C, H, W = x.shape
    Hn, Wn = H // _PATCH, W // _PATCH
    np_ = Hn * Wn
    k_raw = C * _PATCH * _PATCH
    bf16 = jnp.bfloat16
    f32 = jnp.float32

    patches = x.astype(bf16).transpose(0, 2, 3, 1)
    patches = patches.reshape(B, Hn, _PATCH, Wn, _PATCH * C)
    patches = patches.transpose(0, 1, 3, 2, 4)
    patches = patches.reshape(B * np_, k_raw)

    pw = patch_w[:k_raw].reshape(C, _PATCH, _PATCH, _D)
    pw = pw.transpose(1, 2, 0, 3).reshape(k_raw, _D).astype(bf16)

    add_init = jnp.concatenate([
        cls_token.reshape(1, _D) + pos_embed[0, 0:1],
        jnp.zeros((7, _D), f32),
        patch_b + pos_embed[0, 1:]], axis=0)

    wqkv = jnp.concatenate([blk02, blk04, blk06], axis=2).astype(bf16)
    bqkv = jnp.concatenate([blk03, blk05, blk07], axis=2)
    wp = blk08.astype(bf16)
    w12 = jnp.concatenate([blk13, blk15], axis=2).astype(bf16)
    b12 = jnp.concatenate([blk14, blk16], axis=2)
    w3 = blk17.astype(bf16)

    def const(shape):
        nd = len(shape)
        return pl.BlockSpec(shape, lambda g, _n=nd: (0,) * _n)

    args = (patches, pw, add_init,
            blk00, blk01, wqkv, bqkv, wp, blk09, blk10,
            blk11, blk12, w12, b12, w3, blk18, blk19,
            norm_w, norm_b)
    in_specs = [pl.BlockSpec((_BB * np_, k_raw), lambda g: (g, 0))] + \
               [const(a.shape) for a in args[1:]]

    out = pl.pallas_call(
        _fused_kernel,
        grid=(B // _BB,),
        out_shape=jax.ShapeDtypeStruct((B, _D), f32),
        in_specs=in_specs,
        out_specs=pl.BlockSpec((_BB, _D), lambda g: (g, 0)),
        compiler_params=pltpu.CompilerParams(dimension_semantics=("parallel",)),
    )(*args)
    return out


# speedup vs baseline: 1.2934x; 1.0275x over previous
import jax
import jax.numpy as jnp
from jax.experimental import pallas as pl
from jax.experimental.pallas import tpu as pltpu

_PATCH = 14
_D = 128
_DEPTH = 2
_HD = 64
_EPS = 1e-6
_S = 197
_SP = 200
_NP = 196
_BB = 16
_NEG = -1e30


def _ln(v, w, b):
    mu = jnp.mean(v, axis=-1, keepdims=True)
    var = jnp.mean(v * v, axis=-1, keepdims=True) - mu * mu
    return (v - mu) * jax.lax.rsqrt(var + _EPS) * w + b


def _fused_kernel(patches, pw, add_init,
                  ln1w, ln1b, wqkv, bqkv, wp, bp, g1,
                  ln2w, ln2b, w12, b12, w3, b3, g2,
                  nw, nb, o_ref):
    R = _BB * _SP
    bf16 = jnp.bfloat16
    f32 = jnp.float32

    tok = jnp.dot(patches[...], pw[...], preferred_element_type=f32)
    a_cls = add_init[0:1]
    a_pat = add_init[8:8 + _NP]
    zpad = jnp.zeros((_SP - 1 - _NP, _D), f32)
    pieces = []
    for i in range(_BB):
        pieces += [a_cls, tok[i * _NP:(i + 1) * _NP] + a_pat, zpad]
    x = jnp.concatenate(pieces, axis=0)

    lane = jax.lax.broadcasted_iota(jnp.int32, (R, _D), 1)
    head0 = lane < _HD
    col = jax.lax.broadcasted_iota(jnp.int32, (_SP, _SP), 1)
    kvalid = col < _S
    scale = _HD ** -0.5

    for l in range(_DEPTH):
        h = _ln(x, ln1w[l], ln1b[l]).astype(bf16)
        qkv = jnp.dot(h, wqkv[l], preferred_element_type=f32) + bqkv[l]
        q = qkv[:, :_D] * scale
        k = qkv[:, _D:2 * _D]
        v = qkv[:, 2 * _D:]
        qb = q.astype(bf16)
        k0 = jnp.where(head0, k, 0.0).astype(bf16)
        k1 = jnp.where(head0, 0.0, k).astype(bf16)
        v0 = jnp.where(head0, v, 0.0).astype(bf16)
        v1 = jnp.where(head0, 0.0, v).astype(bf16)

        outs = []
        for i in range(_BB):
            a = i * _SP
            qi = qb[a:a + _SP]
            oi = None
            for kk, vv in ((k0, v0), (k1, v1)):
                s = jax.lax.dot_general(
                    qi, kk[a:a + _SP],
                    (((1,), (1,)), ((), ())), preferred_element_type=f32)
                s = jnp.where(kvalid, s, _NEG)
                s = s - jnp.max(s, axis=-1, keepdims=True)
                e = jnp.exp(s)
                p = e * pl.reciprocal(jnp.sum(e, axis=-1, keepdims=True),
                                      approx=True)
                pv = jnp.dot(p.astype(bf16), vv[a:a + _SP],
                             preferred_element_type=f32)
                oi = pv if oi is None else oi + pv
            outs.append(oi)
        attn = jnp.concatenate(outs, axis=0)

        pr = jnp.dot(attn.astype(bf16), wp[l], preferred_element_type=f32) + bp[l]
        x = x + pr * g1[l]

        h2 = _ln(x, ln2w[l], ln2b[l]).astype(bf16)
        x12 = jnp.dot(h2, w12[l], preferred_element_type=f32) + b12[l]
        hw = w12.shape[-1] // 2
        x1 = x12[:, :hw]
        x2 = x12[:, hw:]
        sig = pl.reciprocal(1.0 + jnp.exp(-x1), approx=True)
        hid = ((x1 * sig) * x2).astype(bf16)
        ml = jnp.dot(hid, w3[l], preferred_element_type=f32) + b3[l]
        x = x + ml * g2[l]

    cls = jnp.concatenate([x[i * _SP:i * _SP + 1] for i in range(_BB)], axis=0)
    o_ref[...] = _ln(cls, nw[...], nb[...])


def kernel(x, patch_w, patch_b, cls_token, pos_embed, norm_w, norm_b,
           blk00, blk01, blk02, blk03, blk04, blk05, blk06, blk07, blk08,
           blk09, blk10, blk11, blk12, blk13, blk14, blk15, blk16, blk17,
           blk18, blk19):
    B, C, H, W = x.shape
    Hn, Wn = H // _PATCH, W // _PATCH
    np_ = Hn * Wn
    k_raw = C * _PATCH * _PATCH
    bf16 = jnp.bfloat16
    f32 = jnp.float32

    patches = x.astype(bf16).transpose(0, 2, 3, 1)
    patches = patches.reshape(B, Hn, _PATCH, Wn, _PATCH * C)
    patches = patches.transpose(0, 1, 3, 2, 4)
    patches = patches.reshape(B * np_, k_raw)

    pw = patch_w[:k_raw].reshape(C, _PATCH, _PATCH, _D)
    pw = pw.transpose(1, 2, 0, 3).reshape(k_raw, _D).astype(bf16)

    add_init = jnp.concatenate([
        cls_token.reshape(1, _D) + pos_embed[0, 0:1],
        jnp.zeros((7, _D), f32),
        patch_b + pos_embed[0, 1:]], axis=0)

    wqkv = jnp.concatenate([blk02, blk04, blk06], axis=2).astype(bf16)
    bqkv = jnp.concatenate([blk03, blk05, blk07], axis=2)
    wp = blk08.astype(bf16)
    w12 = jnp.concatenate([blk13, blk15], axis=2).astype(bf16)
    b12 = jnp.concatenate([blk14, blk16], axis=2)
    w3 = blk17.astype(bf16)

    def const(shape):
        nd = len(shape)
        return pl.BlockSpec(shape, lambda g, _n=nd: (0,) * _n)

    args = (patches, pw, add_init,
            blk00, blk01, wqkv, bqkv, wp, blk09, blk10,
            blk11, blk12, w12, b12, w3, blk18, blk19,
            norm_w, norm_b)
    in_specs = [pl.BlockSpec((_BB * np_, k_raw), lambda g: (g, 0))] + \
               [const(a.shape) for a in args[1:]]

    out = pl.pallas_call(
        _fused_kernel,
        grid=(B // _BB,),
        out_shape=jax.ShapeDtypeStruct((B, _D), f32),
        in_specs=in_specs,
        out_specs=pl.BlockSpec((_BB, _D), lambda g: (g, 0)),
        compiler_params=pltpu.CompilerParams(dimension_semantics=("parallel",)),
    )(*args)
    return out
